# baseline (speedup 1.0000x reference)
import functools

import jax
import jax.numpy as jnp
import numpy as np
from jax.experimental import pallas as pl
from jax.experimental.pallas import tpu as pltpu

BS = 2
NA = 128
HW = 4096
C = 256
D_FF = 2048
H = 64
SCALE = 1.0 / np.sqrt(C // 8)

J_BLK = 512

_dot = functools.partial(jnp.dot, preferred_element_type=jnp.float32)


def _decode_ffn_kernel(tok_ref, supp_ref, query_ref,
                       wqa_ref, bqa_ref, wks_ref, bks_ref,
                       wka_ref, bka_ref, wvs_ref, bvs_ref,
                       w1_ref, b1_ref, w2_ref, b2_ref, w3_ref, w1c_ref,
                       out_ref, sa_s, vsc_s, vsum_s, am_s, ka_s, dec_s):
    j = pl.program_id(0)

    @pl.when(j == 0)
    def _():
        for b in range(BS):
            qa = _dot(tok_ref[b], wqa_ref[...]) + bqa_ref[...]
            ks = _dot(supp_ref[b], wks_ref[...]) + bks_ref[...]
            vs = _dot(supp_ref[b], wvs_ref[...]) + bvs_ref[...]
            sa_s[b * NA:(b + 1) * NA, :] = jax.lax.dot_general(
                qa, ks, (((1,), (1,)), ((), ())),
                preferred_element_type=jnp.float32) * SCALE
            vsc_s[:, b * C:(b + 1) * C] = vs
            ka_s[b * NA:(b + 1) * NA, :] = (
                _dot(tok_ref[b], wka_ref[...]) + bka_ref[...])
            if b == 1:
                vsum_s[...] = jnp.broadcast_to(
                    jnp.sum(vs, axis=0, keepdims=True), (8, C))
        sac0 = sa_s[...]
        am_s[0:1, :] = jnp.argmax(sac0[:NA, :], axis=0, keepdims=True)
        am_s[1:2, :] = jnp.argmax(sac0[NA:, :], axis=0, keepdims=True)

    ka0 = ka_s[:NA, :]
    ka1 = ka_s[NA:, :]
    qq0 = _dot(query_ref[0], wqa_ref[...]) + bqa_ref[...]
    qq1 = _dot(query_ref[1], wqa_ref[...]) + bqa_ref[...]
    sq0 = jax.lax.dot_general(qq0, ka0, (((1,), (1,)), ((), ())),
                              preferred_element_type=jnp.float32) * SCALE
    sq1 = jax.lax.dot_general(qq1, ka1, (((1,), (1,)), ((), ())),
                              preferred_element_type=jnp.float32) * SCALE
    sqc = jnp.concatenate([sq0, -sq1], axis=1)
    q0 = jnp.argmax(sq0, axis=1, keepdims=True)
    q1 = jnp.argmax(sq1, axis=1, keepdims=True)
    a0 = am_s[0:1, :]
    a1 = am_s[1:2, :]
    d = _dot(sqc, sa_s[...])
    eq0 = q0 == a0
    eq1 = q1 == a1
    s = jax.nn.sigmoid(d)
    t0 = jnp.where(eq0, jnp.where(eq1, s, 1.0), jnp.where(eq1, 0.0, s))
    t1 = jnp.where(eq0, jnp.where(eq1, 1.0 - s, 0.0),
                   jnp.where(eq1, 1.0, 1.0 - s))
    acc0 = _dot(t0, vsc_s[:, :C])
    acc1 = _dot(t1, vsc_s[:, C:])
    h0 = jnp.maximum(_dot(acc0, w1_ref[...]) + b1_ref[...], 0.0)
    o0 = (_dot(h0, w2_ref[...]) + b2_ref[...]).reshape(J_BLK // 16, 16, C)
    h1 = jnp.maximum(_dot(acc1, w1_ref[...]) + b1_ref[...], 0.0)
    o1 = (_dot(h1, w2_ref[...]) + b2_ref[...]).reshape(J_BLK // 16, 16, C)
    row = pl.multiple_of(j * (J_BLK // 16), J_BLK // 16)
    for r in range(16):
        dec_s[pl.ds(row, J_BLK // 16), r * C:(r + 1) * C] = o0[:, r, :]
        dec_s[pl.ds(C + row, J_BLK // 16), r * C:(r + 1) * C] = o1[:, r, :]

    @pl.when(j == pl.num_programs(0) - 1)
    def _():
        col = jax.lax.broadcasted_iota(jnp.int32, (1, HW), 1) % H
        for b in range(BS):
            x = dec_s[b * C:(b + 1) * C, :]
            y3 = _dot(w3_ref[...], x)
            z = jnp.maximum(_shift_taps(y3, C // 8, col), 0.0)
            y1 = _dot(w1c_ref[...], z)
            out_ref[b] = _shift_taps(y1, 8, col)[:3, :]


def _shift_taps(y, stride, col):
    acc = jnp.zeros((stride, HW), jnp.float32)
    for t in range(9):
        ky, kx = t // 3 - 1, t % 3 - 1
        s = ky * H + kx
        yt = y[t * stride:(t + 1) * stride, :]
        if s > 0:
            sh = jnp.concatenate(
                [yt[:, s:], jnp.zeros((stride, s), jnp.float32)], axis=1)
        elif s < 0:
            sh = jnp.concatenate(
                [jnp.zeros((stride, -s), jnp.float32), yt[:, :HW + s]], axis=1)
        else:
            sh = yt
        if kx == 1:
            sh = jnp.where(col == H - 1, 0.0, sh)
        elif kx == -1:
            sh = jnp.where(col == 0, 0.0, sh)
        acc = acc + sh
    return acc


def kernel(tok_agent, enc_feat_supp, enc_feat_query,
           Wqa, bqa, Wks, bks, Wka, bka, Wvs, bvs,
           W1, b1, W2, b2, conv3_w, conv1_w, *, interpret=False):
    b2d = lambda v: v.reshape(1, -1)
    n_j = HW // J_BLK
    wspec = pl.BlockSpec((C, C), lambda j: (0, 0))
    bspec = pl.BlockSpec((1, C), lambda j: (0, 0))
    ffn_out = pl.pallas_call(
        _decode_ffn_kernel,
        grid=(n_j,),
        in_specs=[
            pl.BlockSpec((BS, NA, C), lambda j: (0, 0, 0)),
            pl.BlockSpec((BS, HW, C), lambda j: (0, 0, 0)),
            pl.BlockSpec((BS, J_BLK, C), lambda j: (0, j, 0)),
            wspec, bspec, wspec, bspec, wspec, bspec, wspec, bspec,
            pl.BlockSpec((C, D_FF), lambda j: (0, 0)),
            pl.BlockSpec((1, D_FF), lambda j: (0, 0)),
            pl.BlockSpec((D_FF, C), lambda j: (0, 0)),
            pl.BlockSpec((1, C), lambda j: (0, 0)),
            pl.BlockSpec((9 * (C // 8), C), lambda j: (0, 0)),
            pl.BlockSpec((9 * 8, C // 8), lambda j: (0, 0)),
        ],
        out_specs=pl.BlockSpec((BS, 3, HW), lambda j: (0, 0, 0)),
        out_shape=jax.ShapeDtypeStruct((BS, 3, HW), jnp.float32),
        scratch_shapes=[
            pltpu.VMEM((BS * NA, HW), jnp.float32),
            pltpu.VMEM((HW, BS * C), jnp.float32),
            pltpu.VMEM((8, C), jnp.float32),
            pltpu.VMEM((8, HW), jnp.int32),
            pltpu.VMEM((BS * NA, C), jnp.float32),
            pltpu.VMEM((BS * C, HW), jnp.float32),
        ],
        compiler_params=pltpu.CompilerParams(
            dimension_semantics=("arbitrary",),
            vmem_limit_bytes=58 * 1024 * 1024,
        ),
        name="decode_ffn",
        interpret=interpret,
    )(tok_agent, enc_feat_supp, enc_feat_query,
      Wqa, b2d(bqa), Wks, b2d(bks), Wka, b2d(bka), Wvs, b2d(bvs),
      W1, b2d(b1), W2, b2d(b2),
      conv3_w.transpose(2, 3, 0, 1).reshape(9 * (C // 8), C),
      jnp.pad(conv1_w.transpose(2, 3, 0, 1).reshape(9, 3, C // 8),
              ((0, 0), (0, 5), (0, 0))).reshape(9 * 8, C // 8))
    return ffn_out.reshape(BS, 3, H, H)


# speedup vs baseline: 1.1305x; 1.1305x over previous
import functools

import jax
import jax.numpy as jnp
import numpy as np
from jax.experimental import pallas as pl
from jax.experimental.pallas import tpu as pltpu

BS = 2
NA = 128
HW = 4096
C = 256
D_FF = 2048
H = 64
SCALE = 1.0 / np.sqrt(C // 8)

J_BLK = 512

_dot = functools.partial(jnp.dot, preferred_element_type=jnp.float32)


def _decode_ffn_kernel(tok_ref, supp_ref, query_ref,
                       wqa_ref, bqa_ref, wks_ref, bks_ref,
                       wka_ref, bka_ref, wvs_ref, bvs_ref,
                       w1_ref, b1_ref, w2_ref, b2_ref, w3_ref, w1c_ref,
                       out_ref, sa_s, vsc_s, vsum_s, am_s, ka_s, dec_s):
    j = pl.program_id(0)

    @pl.when(j == 0)
    def _():
        for b in range(BS):
            qa = _dot(tok_ref[b], wqa_ref[...]) + bqa_ref[...]
            ks = _dot(supp_ref[b], wks_ref[...]) + bks_ref[...]
            vs = _dot(supp_ref[b], wvs_ref[...]) + bvs_ref[...]
            sa_s[b * NA:(b + 1) * NA, :] = jax.lax.dot_general(
                qa, ks, (((1,), (1,)), ((), ())),
                preferred_element_type=jnp.float32) * SCALE
            vsc_s[:, b * C:(b + 1) * C] = vs
            ka_s[b * NA:(b + 1) * NA, :] = (
                _dot(tok_ref[b], wka_ref[...]) + bka_ref[...])
            if b == 1:
                vsum_s[...] = jnp.broadcast_to(
                    jnp.sum(vs, axis=0, keepdims=True), (8, C))
        sac0 = sa_s[...]
        am_s[0:1, :] = jnp.argmax(sac0[:NA, :], axis=0, keepdims=True)
        am_s[1:2, :] = jnp.argmax(sac0[NA:, :], axis=0, keepdims=True)

    ka0 = ka_s[:NA, :]
    ka1 = ka_s[NA:, :]
    qq0 = _dot(query_ref[0], wqa_ref[...]) + bqa_ref[...]
    qq1 = _dot(query_ref[1], wqa_ref[...]) + bqa_ref[...]
    sq0 = jax.lax.dot_general(qq0, ka0, (((1,), (1,)), ((), ())),
                              preferred_element_type=jnp.float32) * SCALE
    sq1 = jax.lax.dot_general(qq1, ka1, (((1,), (1,)), ((), ())),
                              preferred_element_type=jnp.float32) * SCALE
    sqc = jnp.concatenate([sq0, -sq1], axis=1)
    q0 = jnp.argmax(sq0, axis=1, keepdims=True)
    q1 = jnp.argmax(sq1, axis=1, keepdims=True)
    ab = jnp.zeros((J_BLK, BS * C), jnp.float32)
    for ic in range(2):
        sl = slice(ic * (HW // 2), (ic + 1) * (HW // 2))
        a0 = am_s[0:1, sl]
        a1 = am_s[1:2, sl]
        d = _dot(sqc, sa_s[:, sl])
        eq0 = q0 == a0
        eq1 = q1 == a1
        s = jax.nn.sigmoid(d)
        t0 = jnp.where(eq0, jnp.where(eq1, s, 1.0), jnp.where(eq1, 0.0, s))
        ab = ab + _dot(t0, vsc_s[sl, :])
    acc0 = ab[:, :C]
    acc1 = vsum_s[0:1, :] - ab[:, C:]
    h0 = jnp.maximum(_dot(acc0, w1_ref[...]) + b1_ref[...], 0.0)
    o0 = (_dot(h0, w2_ref[...]) + b2_ref[...]).reshape(J_BLK // 16, 16, C)
    h1 = jnp.maximum(_dot(acc1, w1_ref[...]) + b1_ref[...], 0.0)
    o1 = (_dot(h1, w2_ref[...]) + b2_ref[...]).reshape(J_BLK // 16, 16, C)
    row = pl.multiple_of(j * (J_BLK // 16), J_BLK // 16)
    for r in range(16):
        dec_s[pl.ds(row, J_BLK // 16), r * C:(r + 1) * C] = o0[:, r, :]
        dec_s[pl.ds(C + row, J_BLK // 16), r * C:(r + 1) * C] = o1[:, r, :]

    @pl.when(j == pl.num_programs(0) - 1)
    def _():
        col = jax.lax.broadcasted_iota(jnp.int32, (1, HW), 1) % H
        for b in range(BS):
            x = dec_s[b * C:(b + 1) * C, :]
            y3 = _dot(w3_ref[...], x)
            z = jnp.maximum(_shift_taps(y3, C // 8, col), 0.0)
            y1 = _dot(w1c_ref[...], z)
            out_ref[b] = _shift_taps(y1, 8, col)[:3, :]


def _shift_taps(y, stride, col):
    acc = jnp.zeros((stride, HW), jnp.float32)
    for t in range(9):
        ky, kx = t // 3 - 1, t % 3 - 1
        s = ky * H + kx
        yt = y[t * stride:(t + 1) * stride, :]
        if s > 0:
            sh = jnp.concatenate(
                [yt[:, s:], jnp.zeros((stride, s), jnp.float32)], axis=1)
        elif s < 0:
            sh = jnp.concatenate(
                [jnp.zeros((stride, -s), jnp.float32), yt[:, :HW + s]], axis=1)
        else:
            sh = yt
        if kx == 1:
            sh = jnp.where(col == H - 1, 0.0, sh)
        elif kx == -1:
            sh = jnp.where(col == 0, 0.0, sh)
        acc = acc + sh
    return acc


def kernel(tok_agent, enc_feat_supp, enc_feat_query,
           Wqa, bqa, Wks, bks, Wka, bka, Wvs, bvs,
           W1, b1, W2, b2, conv3_w, conv1_w, *, interpret=False):
    b2d = lambda v: v.reshape(1, -1)
    n_j = HW // J_BLK
    wspec = pl.BlockSpec((C, C), lambda j: (0, 0))
    bspec = pl.BlockSpec((1, C), lambda j: (0, 0))
    ffn_out = pl.pallas_call(
        _decode_ffn_kernel,
        grid=(n_j,),
        in_specs=[
            pl.BlockSpec((BS, NA, C), lambda j: (0, 0, 0)),
            pl.BlockSpec((BS, HW, C), lambda j: (0, 0, 0)),
            pl.BlockSpec((BS, J_BLK, C), lambda j: (0, j, 0)),
            wspec, bspec, wspec, bspec, wspec, bspec, wspec, bspec,
            pl.BlockSpec((C, D_FF), lambda j: (0, 0)),
            pl.BlockSpec((1, D_FF), lambda j: (0, 0)),
            pl.BlockSpec((D_FF, C), lambda j: (0, 0)),
            pl.BlockSpec((1, C), lambda j: (0, 0)),
            pl.BlockSpec((9 * (C // 8), C), lambda j: (0, 0)),
            pl.BlockSpec((9 * 8, C // 8), lambda j: (0, 0)),
        ],
        out_specs=pl.BlockSpec((BS, 3, HW), lambda j: (0, 0, 0)),
        out_shape=jax.ShapeDtypeStruct((BS, 3, HW), jnp.float32),
        scratch_shapes=[
            pltpu.VMEM((BS * NA, HW), jnp.float32),
            pltpu.VMEM((HW, BS * C), jnp.float32),
            pltpu.VMEM((8, C), jnp.float32),
            pltpu.VMEM((8, HW), jnp.int32),
            pltpu.VMEM((BS * NA, C), jnp.float32),
            pltpu.VMEM((BS * C, HW), jnp.float32),
        ],
        compiler_params=pltpu.CompilerParams(
            dimension_semantics=("arbitrary",),
            vmem_limit_bytes=58 * 1024 * 1024,
        ),
        name="decode_ffn",
        interpret=interpret,
    )(tok_agent, enc_feat_supp, enc_feat_query,
      Wqa, b2d(bqa), Wks, b2d(bks), Wka, b2d(bka), Wvs, b2d(bvs),
      W1, b2d(b1), W2, b2d(b2),
      conv3_w.transpose(2, 3, 0, 1).reshape(9 * (C // 8), C),
      jnp.pad(conv1_w.transpose(2, 3, 0, 1).reshape(9, 3, C // 8),
              ((0, 0), (0, 5), (0, 0))).reshape(9 * 8, C // 8))
    return ffn_out.reshape(BS, 3, H, H)


# speedup vs baseline: 1.1356x; 1.0045x over previous
import functools

import jax
import jax.numpy as jnp
import numpy as np
from jax.experimental import pallas as pl
from jax.experimental.pallas import tpu as pltpu

BS = 2
NA = 128
HW = 4096
C = 256
D_FF = 2048
H = 64
SCALE = 1.0 / np.sqrt(C // 8)

J_BLK = 512

_dot = functools.partial(jnp.dot, preferred_element_type=jnp.float32)


def _decode_ffn_kernel(tok_ref, supp_ref, query_ref,
                       wqa_ref, bqa_ref, wks_ref, bks_ref,
                       wka_ref, bka_ref, wvs_ref, bvs_ref,
                       w1_ref, b1_ref, w2_ref, b2_ref, w3_ref, w1c_ref,
                       out_ref, sa_s, vsc_s, vsum_s, am_s, ka_s, dec_s):
    j = pl.program_id(0)

    @pl.when(j == 0)
    def _():
        for b in range(BS):
            qa = _dot(tok_ref[b], wqa_ref[...]) + bqa_ref[...]
            ks = _dot(supp_ref[b], wks_ref[...]) + bks_ref[...]
            vs = _dot(supp_ref[b], wvs_ref[...]) + bvs_ref[...]
            sa_s[b * NA:(b + 1) * NA, :] = jax.lax.dot_general(
                qa, ks, (((1,), (1,)), ((), ())),
                preferred_element_type=jnp.float32) * SCALE
            vsc_s[:, b * C:(b + 1) * C] = vs
            ka_s[b * NA:(b + 1) * NA, :] = (
                _dot(tok_ref[b], wka_ref[...]) + bka_ref[...])
            if b == 1:
                vsum_s[...] = jnp.broadcast_to(
                    jnp.sum(vs, axis=0, keepdims=True), (8, C))
        sac0 = sa_s[...]
        am_s[0:1, :] = jnp.argmax(sac0[:NA, :], axis=0, keepdims=True)
        am_s[1:2, :] = jnp.argmax(sac0[NA:, :], axis=0, keepdims=True)

    ka0 = ka_s[:NA, :]
    ka1 = ka_s[NA:, :]
    qq0 = _dot(query_ref[0], wqa_ref[...]) + bqa_ref[...]
    qq1 = _dot(query_ref[1], wqa_ref[...]) + bqa_ref[...]
    sq0 = jax.lax.dot_general(qq0, ka0, (((1,), (1,)), ((), ())),
                              preferred_element_type=jnp.float32) * SCALE
    sq1 = jax.lax.dot_general(qq1, ka1, (((1,), (1,)), ((), ())),
                              preferred_element_type=jnp.float32) * SCALE
    sqc = jnp.concatenate([sq0, -sq1], axis=1)
    q0 = jnp.argmax(sq0, axis=1, keepdims=True)
    q1 = jnp.argmax(sq1, axis=1, keepdims=True)
    ab = jnp.zeros((J_BLK, BS * C), jnp.float32)
    for ic in range(4):
        sl = slice(ic * (HW // 4), (ic + 1) * (HW // 4))
        a0 = am_s[0:1, sl]
        a1 = am_s[1:2, sl]
        d = _dot(sqc, sa_s[:, sl])
        eq0 = q0 == a0
        eq1 = q1 == a1
        s = jax.nn.sigmoid(d)
        t0 = jnp.where(eq0, jnp.where(eq1, s, 1.0), jnp.where(eq1, 0.0, s))
        ab = ab + _dot(t0, vsc_s[sl, :])
    acc0 = ab[:, :C]
    acc1 = vsum_s[0:1, :] - ab[:, C:]
    h0 = jnp.maximum(_dot(acc0, w1_ref[...]) + b1_ref[...], 0.0)
    o0 = (_dot(h0, w2_ref[...]) + b2_ref[...]).reshape(J_BLK // 16, 16, C)
    h1 = jnp.maximum(_dot(acc1, w1_ref[...]) + b1_ref[...], 0.0)
    o1 = (_dot(h1, w2_ref[...]) + b2_ref[...]).reshape(J_BLK // 16, 16, C)
    row = pl.multiple_of(j * (J_BLK // 16), J_BLK // 16)
    for r in range(16):
        dec_s[pl.ds(row, J_BLK // 16), r * C:(r + 1) * C] = o0[:, r, :]
        dec_s[pl.ds(C + row, J_BLK // 16), r * C:(r + 1) * C] = o1[:, r, :]

    @pl.when(j == pl.num_programs(0) - 1)
    def _():
        col = jax.lax.broadcasted_iota(jnp.int32, (1, HW), 1) % H
        for b in range(BS):
            x = dec_s[b * C:(b + 1) * C, :]
            y3 = _dot(w3_ref[...], x)
            z = jnp.maximum(_shift_taps(y3, C // 8, col), 0.0)
            y1 = _dot(w1c_ref[...], z)
            out_ref[b] = _shift_taps(y1, 8, col)[:3, :]


def _shift_taps(y, stride, col):
    acc = jnp.zeros((stride, HW), jnp.float32)
    for t in range(9):
        ky, kx = t // 3 - 1, t % 3 - 1
        s = ky * H + kx
        yt = y[t * stride:(t + 1) * stride, :]
        if s > 0:
            sh = jnp.concatenate(
                [yt[:, s:], jnp.zeros((stride, s), jnp.float32)], axis=1)
        elif s < 0:
            sh = jnp.concatenate(
                [jnp.zeros((stride, -s), jnp.float32), yt[:, :HW + s]], axis=1)
        else:
            sh = yt
        if kx == 1:
            sh = jnp.where(col == H - 1, 0.0, sh)
        elif kx == -1:
            sh = jnp.where(col == 0, 0.0, sh)
        acc = acc + sh
    return acc


def kernel(tok_agent, enc_feat_supp, enc_feat_query,
           Wqa, bqa, Wks, bks, Wka, bka, Wvs, bvs,
           W1, b1, W2, b2, conv3_w, conv1_w, *, interpret=False):
    b2d = lambda v: v.reshape(1, -1)
    n_j = HW // J_BLK
    wspec = pl.BlockSpec((C, C), lambda j: (0, 0))
    bspec = pl.BlockSpec((1, C), lambda j: (0, 0))
    ffn_out = pl.pallas_call(
        _decode_ffn_kernel,
        grid=(n_j,),
        in_specs=[
            pl.BlockSpec((BS, NA, C), lambda j: (0, 0, 0)),
            pl.BlockSpec((BS, HW, C), lambda j: (0, 0, 0)),
            pl.BlockSpec((BS, J_BLK, C), lambda j: (0, j, 0)),
            wspec, bspec, wspec, bspec, wspec, bspec, wspec, bspec,
            pl.BlockSpec((C, D_FF), lambda j: (0, 0)),
            pl.BlockSpec((1, D_FF), lambda j: (0, 0)),
            pl.BlockSpec((D_FF, C), lambda j: (0, 0)),
            pl.BlockSpec((1, C), lambda j: (0, 0)),
            pl.BlockSpec((9 * (C // 8), C), lambda j: (0, 0)),
            pl.BlockSpec((9 * 8, C // 8), lambda j: (0, 0)),
        ],
        out_specs=pl.BlockSpec((BS, 3, HW), lambda j: (0, 0, 0)),
        out_shape=jax.ShapeDtypeStruct((BS, 3, HW), jnp.float32),
        scratch_shapes=[
            pltpu.VMEM((BS * NA, HW), jnp.float32),
            pltpu.VMEM((HW, BS * C), jnp.float32),
            pltpu.VMEM((8, C), jnp.float32),
            pltpu.VMEM((8, HW), jnp.int32),
            pltpu.VMEM((BS * NA, C), jnp.float32),
            pltpu.VMEM((BS * C, HW), jnp.float32),
        ],
        compiler_params=pltpu.CompilerParams(
            dimension_semantics=("arbitrary",),
            vmem_limit_bytes=58 * 1024 * 1024,
        ),
        name="decode_ffn",
        interpret=interpret,
    )(tok_agent, enc_feat_supp, enc_feat_query,
      Wqa, b2d(bqa), Wks, b2d(bks), Wka, b2d(bka), Wvs, b2d(bvs),
      W1, b2d(b1), W2, b2d(b2),
      conv3_w.transpose(2, 3, 0, 1).reshape(9 * (C // 8), C),
      jnp.pad(conv1_w.transpose(2, 3, 0, 1).reshape(9, 3, C // 8),
              ((0, 0), (0, 5), (0, 0))).reshape(9 * 8, C // 8))
    return ffn_out.reshape(BS, 3, H, H)


# speedup vs baseline: 1.1543x; 1.0165x over previous
import functools

import jax
import jax.numpy as jnp
import numpy as np
from jax.experimental import pallas as pl
from jax.experimental.pallas import tpu as pltpu

BS = 2
NA = 128
HW = 4096
C = 256
D_FF = 2048
H = 64
SCALE = 1.0 / np.sqrt(C // 8)

J_BLK = 512

_dot = functools.partial(jnp.dot, preferred_element_type=jnp.float32)


def _decode_ffn_kernel(tok_ref, supp_ref, query_ref,
                       wqa_ref, bqa_ref, wks_ref, bks_ref,
                       wka_ref, bka_ref, wvs_ref, bvs_ref,
                       w1_ref, b1_ref, w2_ref, b2_ref, w3_ref, w1c_ref,
                       out_ref, sa_s, vsc_s, vsum_s, am_s, ka_s, dec_s):
    j = pl.program_id(0)

    @pl.when(j == 0)
    def _():
        for b in range(BS):
            qa = _dot(tok_ref[b], wqa_ref[...]) + bqa_ref[...]
            ks = _dot(supp_ref[b], wks_ref[...]) + bks_ref[...]
            vs = _dot(supp_ref[b], wvs_ref[...]) + bvs_ref[...]
            sa_s[b * NA:(b + 1) * NA, :] = jax.lax.dot_general(
                qa, ks, (((1,), (1,)), ((), ())),
                preferred_element_type=jnp.float32) * SCALE
            vsc_s[:, b * C:(b + 1) * C] = vs
            ka_s[b * NA:(b + 1) * NA, :] = (
                _dot(tok_ref[b], wka_ref[...]) + bka_ref[...])
            if b == 1:
                vsum_s[...] = jnp.broadcast_to(
                    jnp.sum(vs, axis=0, keepdims=True), (8, C))
        sac0 = sa_s[...]
        am_s[0:1, :] = jnp.argmax(sac0[:NA, :], axis=0, keepdims=True)
        am_s[1:2, :] = jnp.argmax(sac0[NA:, :], axis=0, keepdims=True)

    ka0 = ka_s[:NA, :]
    ka1 = ka_s[NA:, :]
    qq0 = _dot(query_ref[0], wqa_ref[...]) + bqa_ref[...]
    qq1 = _dot(query_ref[1], wqa_ref[...]) + bqa_ref[...]
    sq0 = jax.lax.dot_general(qq0, ka0, (((1,), (1,)), ((), ())),
                              preferred_element_type=jnp.float32) * SCALE
    sq1 = jax.lax.dot_general(qq1, ka1, (((1,), (1,)), ((), ())),
                              preferred_element_type=jnp.float32) * SCALE
    sqc = jnp.concatenate([sq0, -sq1], axis=1)
    q0 = jnp.argmax(sq0, axis=1, keepdims=True)
    q1 = jnp.argmax(sq1, axis=1, keepdims=True)
    ab = jnp.zeros((J_BLK, BS * C), jnp.float32)
    for ic in range(8):
        sl = slice(ic * (HW // 8), (ic + 1) * (HW // 8))
        a0 = am_s[0:1, sl]
        a1 = am_s[1:2, sl]
        d = _dot(sqc, sa_s[:, sl])
        eq0 = q0 == a0
        eq1 = q1 == a1
        s = jax.nn.sigmoid(d)
        t0 = jnp.where(eq0, jnp.where(eq1, s, 1.0), jnp.where(eq1, 0.0, s))
        ab = ab + _dot(t0, vsc_s[sl, :])
    acc0 = ab[:, :C]
    acc1 = vsum_s[0:1, :] - ab[:, C:]
    h0 = jnp.maximum(_dot(acc0, w1_ref[...]) + b1_ref[...], 0.0)
    o0 = (_dot(h0, w2_ref[...]) + b2_ref[...]).reshape(J_BLK // 16, 16, C)
    h1 = jnp.maximum(_dot(acc1, w1_ref[...]) + b1_ref[...], 0.0)
    o1 = (_dot(h1, w2_ref[...]) + b2_ref[...]).reshape(J_BLK // 16, 16, C)
    row = pl.multiple_of(j * (J_BLK // 16), J_BLK // 16)
    for r in range(16):
        dec_s[pl.ds(row, J_BLK // 16), r * C:(r + 1) * C] = o0[:, r, :]
        dec_s[pl.ds(C + row, J_BLK // 16), r * C:(r + 1) * C] = o1[:, r, :]

    @pl.when(j == pl.num_programs(0) - 1)
    def _():
        col = jax.lax.broadcasted_iota(jnp.int32, (1, HW), 1) % H
        for b in range(BS):
            x = dec_s[b * C:(b + 1) * C, :]
            y3 = _dot(w3_ref[...], x)
            z = jnp.maximum(_shift_taps(y3, C // 8, col), 0.0)
            y1 = _dot(w1c_ref[...], z)
            out_ref[b] = _shift_taps(y1, 8, col)[:3, :]


def _shift_taps(y, stride, col):
    acc = jnp.zeros((stride, HW), jnp.float32)
    for t in range(9):
        ky, kx = t // 3 - 1, t % 3 - 1
        s = ky * H + kx
        yt = y[t * stride:(t + 1) * stride, :]
        if s > 0:
            sh = jnp.concatenate(
                [yt[:, s:], jnp.zeros((stride, s), jnp.float32)], axis=1)
        elif s < 0:
            sh = jnp.concatenate(
                [jnp.zeros((stride, -s), jnp.float32), yt[:, :HW + s]], axis=1)
        else:
            sh = yt
        if kx == 1:
            sh = jnp.where(col == H - 1, 0.0, sh)
        elif kx == -1:
            sh = jnp.where(col == 0, 0.0, sh)
        acc = acc + sh
    return acc


def kernel(tok_agent, enc_feat_supp, enc_feat_query,
           Wqa, bqa, Wks, bks, Wka, bka, Wvs, bvs,
           W1, b1, W2, b2, conv3_w, conv1_w, *, interpret=False):
    b2d = lambda v: v.reshape(1, -1)
    n_j = HW // J_BLK
    wspec = pl.BlockSpec((C, C), lambda j: (0, 0))
    bspec = pl.BlockSpec((1, C), lambda j: (0, 0))
    ffn_out = pl.pallas_call(
        _decode_ffn_kernel,
        grid=(n_j,),
        in_specs=[
            pl.BlockSpec((BS, NA, C), lambda j: (0, 0, 0)),
            pl.BlockSpec((BS, HW, C), lambda j: (0, 0, 0)),
            pl.BlockSpec((BS, J_BLK, C), lambda j: (0, j, 0)),
            wspec, bspec, wspec, bspec, wspec, bspec, wspec, bspec,
            pl.BlockSpec((C, D_FF), lambda j: (0, 0)),
            pl.BlockSpec((1, D_FF), lambda j: (0, 0)),
            pl.BlockSpec((D_FF, C), lambda j: (0, 0)),
            pl.BlockSpec((1, C), lambda j: (0, 0)),
            pl.BlockSpec((9 * (C // 8), C), lambda j: (0, 0)),
            pl.BlockSpec((9 * 8, C // 8), lambda j: (0, 0)),
        ],
        out_specs=pl.BlockSpec((BS, 3, HW), lambda j: (0, 0, 0)),
        out_shape=jax.ShapeDtypeStruct((BS, 3, HW), jnp.float32),
        scratch_shapes=[
            pltpu.VMEM((BS * NA, HW), jnp.float32),
            pltpu.VMEM((HW, BS * C), jnp.float32),
            pltpu.VMEM((8, C), jnp.float32),
            pltpu.VMEM((8, HW), jnp.int32),
            pltpu.VMEM((BS * NA, C), jnp.float32),
            pltpu.VMEM((BS * C, HW), jnp.float32),
        ],
        compiler_params=pltpu.CompilerParams(
            dimension_semantics=("arbitrary",),
            vmem_limit_bytes=58 * 1024 * 1024,
        ),
        name="decode_ffn",
        interpret=interpret,
    )(tok_agent, enc_feat_supp, enc_feat_query,
      Wqa, b2d(bqa), Wks, b2d(bks), Wka, b2d(bka), Wvs, b2d(bvs),
      W1, b2d(b1), W2, b2d(b2),
      conv3_w.transpose(2, 3, 0, 1).reshape(9 * (C // 8), C),
      jnp.pad(conv1_w.transpose(2, 3, 0, 1).reshape(9, 3, C // 8),
              ((0, 0), (0, 5), (0, 0))).reshape(9 * 8, C // 8))
    return ffn_out.reshape(BS, 3, H, H)
